# Initial kernel scaffold; baseline (speedup 1.0000x reference)
import jax
import jax.numpy as jnp
from jax.experimental import pallas as pl


def kernel(x, category, W0, b0, g0, beta0, W1, b1, g1, beta1, W2, b2, g2, beta2):
    raise NotImplementedError("write your pallas kernel here")


# speedup vs baseline: 1.5345x; 1.5345x over previous
import functools

import jax
import jax.numpy as jnp
from jax import lax
from jax.experimental import pallas as pl
from jax.experimental.pallas import tpu as pltpu
from jax.experimental.pallas import tpu_sc as plsc

N = 100000
C = 10000
D = 128
H = 64

R = 1000
GRID = N // R
NT = 32
CW = 313
C_PAD = NT * CW
CH = 256
B_PW = 3136
NP = NT * B_PW
GCH = 112
G_ITERS = B_PW // GCH

_F32 = jnp.float32


def _rsqrt_precise(v):
    r = lax.rsqrt(v)
    return r * (1.5 - 0.5 * v * r * r)


def _ln_relu(z, gm, bt):
    mu = jnp.mean(z, axis=-1, keepdims=True)
    zc = z - mu
    var = jnp.mean(zc * zc, axis=-1, keepdims=True)
    h = zc * _rsqrt_precise(var + 1e-5) * gm + bt
    return jnp.maximum(h, 0.0)


def _dot(a, b):
    return lax.dot_general(a, b, (((1,), (0,)), ((), ())),
                           preferred_element_type=_F32)


def _mlp0_body(x_ref, cat_ref, w_ref, b_ref, gm_ref, bt_ref, h_ref, s_ref):
    i = pl.program_id(0)
    z = _dot(x_ref[...], w_ref[...]) + b_ref[...]
    h_ref[...] = _ln_relu(z, gm_ref[...], bt_ref[...])
    cat = cat_ref[0]
    th = lax.broadcasted_iota(jnp.int32, (1, 128), 1) * CW
    cmp = (cat < th).astype(_F32)
    ssum = jnp.sum(cmp, axis=0, keepdims=True)

    @pl.when(i == 0)
    def _():
        s_ref[...] = jnp.zeros_like(s_ref)
    s_ref[...] += ssum


def _mlp_mid_body(h_ref, g_ref, w_ref, b_ref, gm_ref, bt_ref, o_ref):
    z = _dot(h_ref[...], w_ref[...]) + g_ref[...] + b_ref[...]
    o_ref[...] = _ln_relu(z, gm_ref[...], bt_ref[...])


def _mlp_last_body(h_ref, g_ref, w_ref, b_ref, gm_ref, bt_ref, o_ref, ss_ref):
    i = pl.program_id(0)
    z = _dot(h_ref[...], w_ref[...]) + g_ref[...] + b_ref[...]
    h = _ln_relu(z, gm_ref[...], bt_ref[...])
    o_ref[...] = h

    @pl.when(i == 0)
    def _():
        ss_ref[...] = jnp.zeros_like(ss_ref)
    ss_ref[...] += jnp.sum(h * h, axis=0, keepdims=True)


def _tabmm_body(a_ref, w_ref, p_ref):
    p_ref[...] = _dot(a_ref[...], w_ref[...])


def _colsq_body(g_ref, ss_ref):
    i = pl.program_id(0)

    @pl.when(i == 0)
    def _():
        ss_ref[...] = jnp.zeros_like(ss_ref)
    g = g_ref[...]
    ss_ref[...] += jnp.sum(g * g, axis=0, keepdims=True)


def _final_body(h_ref, g_ref, ssh_ref, ssg_ref, y_ref):
    ih = _rsqrt_precise(ssh_ref[...] + 1e-30)
    ig = _rsqrt_precise(ssg_ref[...] + 1e-30)
    y_ref[...] = jnp.concatenate([h_ref[...] * ih, g_ref[...] * ig], axis=1)


def _row_spec(cols):
    return pl.BlockSpec((R, cols), lambda i: (i, 0))


def _const_spec(shape):
    nd = len(shape)
    return pl.BlockSpec(shape, lambda i: (0,) * nd)


def _tc_mlp0(x, cat3, w, b, gm, bt):
    return pl.pallas_call(
        _mlp0_body,
        grid=(GRID,),
        in_specs=[
            _row_spec(128),
            pl.BlockSpec((1, R, 1), lambda i: (i, 0, 0)),
            _const_spec((128, 64)),
            _const_spec((1, 64)),
            _const_spec((1, 64)),
            _const_spec((1, 64)),
        ],
        out_specs=[_row_spec(64), _const_spec((1, 128))],
        out_shape=[
            jax.ShapeDtypeStruct((NP, 64), _F32),
            jax.ShapeDtypeStruct((1, 128), _F32),
        ],
    )(x, cat3, w, b, gm, bt)


def _tc_mid(h, g, w, b, gm, bt):
    return pl.pallas_call(
        _mlp_mid_body,
        grid=(GRID,),
        in_specs=[
            _row_spec(64), _row_spec(64),
            _const_spec((64, 64)),
            _const_spec((1, 64)), _const_spec((1, 64)), _const_spec((1, 64)),
        ],
        out_specs=_row_spec(64),
        out_shape=jax.ShapeDtypeStruct((NP, 64), _F32),
    )(h, g, w, b, gm, bt)


def _tc_last(h, g, w, b, gm, bt):
    return pl.pallas_call(
        _mlp_last_body,
        grid=(GRID,),
        in_specs=[
            _row_spec(64), _row_spec(64),
            _const_spec((64, 64)),
            _const_spec((1, 64)), _const_spec((1, 64)), _const_spec((1, 64)),
        ],
        out_specs=[_row_spec(64), _const_spec((1, 64))],
        out_shape=[
            jax.ShapeDtypeStruct((NP, 64), _F32),
            jax.ShapeDtypeStruct((1, 64), _F32),
        ],
    )(h, g, w, b, gm, bt)


def _tc_tabmm(a, w):
    rp = C_PAD // 4
    return pl.pallas_call(
        _tabmm_body,
        grid=(4,),
        in_specs=[
            pl.BlockSpec((rp, 64), lambda i: (i, 0)),
            _const_spec((64, 64)),
        ],
        out_specs=pl.BlockSpec((rp, 64), lambda i: (i, 0)),
        out_shape=jax.ShapeDtypeStruct((C_PAD, 64), _F32),
    )(a, w)


def _tc_colsq(g):
    return pl.pallas_call(
        _colsq_body,
        grid=(GRID,),
        in_specs=[_row_spec(64)],
        out_specs=_const_spec((1, 64)),
        out_shape=jax.ShapeDtypeStruct((1, 64), _F32),
    )(g)


def _tc_final(h, g, ssh, ssg):
    return pl.pallas_call(
        _final_body,
        grid=(GRID,),
        in_specs=[
            _row_spec(64), _row_spec(64),
            _const_spec((1, 64)), _const_spec((1, 64)),
        ],
        out_specs=_row_spec(128),
        out_shape=jax.ShapeDtypeStruct((N, 128), _F32),
    )(h, g, ssh, ssg)


def _sc_mesh():
    return plsc.VectorSubcoreMesh(core_axis_name="c", subcore_axis_name="s",
                                  num_cores=2, num_subcores=16)


def _segmax_body(h_hbm, cat_hbm, st_hbm, a_hbm, sv, catbuf, hbuf, abuf):
    wid = lax.axis_index("s") * 2 + lax.axis_index("c")
    pltpu.sync_copy(st_hbm.at[pl.ds(0, 48)], sv.at[pl.ds(0, 48)])
    sw = sv[pl.ds(wid, 16)]
    start = sw[0]
    end = sw[1]
    c_lo = wid * CW

    zv = jnp.zeros((16,), _F32)

    def zb(i, _):
        abuf[pl.ds(i * 16, 16)] = zv
        return 0
    lax.fori_loop(0, CW * 4, zb, 0)

    a0 = (start // 8) * 8
    total = end - a0
    nch = (total + CH - 1) // CH

    def chunk(k, carry):
        r0 = a0 + k * CH
        pltpu.sync_copy(cat_hbm.at[pl.ds(r0, CH)], catbuf.at[pl.ds(0, CH)])
        pltpu.sync_copy(h_hbm.at[pl.ds(r0 * 64, CH * 64)], hbuf)
        lo = jnp.maximum(start - r0, 0)
        hi = jnp.minimum(end - r0, CH)

        def row(r, rc):
            prev, q0, q1, q2, q3 = rc
            c = catbuf[pl.ds(r, 16)][0]
            keep = jnp.where(c != prev, 0.0, 1.0).astype(_F32)
            rb = r * 64
            q0 = jnp.maximum(hbuf[pl.ds(rb, 16)], q0 * keep)
            q1 = jnp.maximum(hbuf[pl.ds(rb + 16, 16)], q1 * keep)
            q2 = jnp.maximum(hbuf[pl.ds(rb + 32, 16)], q2 * keep)
            q3 = jnp.maximum(hbuf[pl.ds(rb + 48, 16)], q3 * keep)
            ab = (c - c_lo) * 64
            abuf[pl.ds(ab, 16)] = q0
            abuf[pl.ds(ab + 16, 16)] = q1
            abuf[pl.ds(ab + 32, 16)] = q2
            abuf[pl.ds(ab + 48, 16)] = q3
            return (c, q0, q1, q2, q3)

        return lax.fori_loop(lo, hi, row, carry)

    lax.fori_loop(0, nch, chunk, (jnp.int32(-1), zv, zv, zv, zv))
    pltpu.sync_copy(abuf, a_hbm.at[pl.ds(c_lo * 64, CW * 64)])


def _sc_segmax(h_flat, cat, starts):
    run = functools.partial(
        pl.kernel,
        out_type=jax.ShapeDtypeStruct((C_PAD * 64,), _F32),
        mesh=_sc_mesh(),
        scratch_types=[
            pltpu.VMEM((64,), jnp.int32),
            pltpu.VMEM((CH + 16,), jnp.int32),
            pltpu.VMEM((CH * 64,), _F32),
            pltpu.VMEM((CW * 64,), _F32),
        ],
    )(_segmax_body)
    return run(h_flat, cat, starts)


def _gather_body(p_hbm, cat_hbm, g_hbm, idxbuf, rowbuf, sem):
    wid = lax.axis_index("s") * 2 + lax.axis_index("c")
    base = wid * B_PW

    def chunk(k, _):
        r0 = base + k * GCH
        pltpu.sync_copy(cat_hbm.at[pl.ds(r0, GCH)], idxbuf)
        pltpu.async_copy(p_hbm.at[idxbuf], rowbuf, sem).wait()
        pltpu.sync_copy(rowbuf, g_hbm.at[pl.ds(r0, GCH)])
        return 0
    lax.fori_loop(0, G_ITERS, chunk, 0)


def _sc_gather(p, cat):
    run = functools.partial(
        pl.kernel,
        out_type=jax.ShapeDtypeStruct((NP, 64), _F32),
        mesh=_sc_mesh(),
        compiler_params=pltpu.CompilerParams(use_tc_tiling_on_sc=False),
        scratch_types=[
            pltpu.VMEM((GCH,), jnp.int32),
            pltpu.VMEM((GCH, 64), _F32),
            pltpu.SemaphoreType.DMA,
        ],
    )(_gather_body)
    return run(p, cat)


def kernel(x, category, W0, b0, g0, beta0, W1, b1, g1, beta1,
           W2, b2, g2, beta2):
    cat = category.astype(jnp.int32)
    cat_pad = jnp.pad(cat, (0, NP - N))
    cat3 = cat.reshape(GRID, R, 1)

    def r2(v):
        return v.reshape(1, 64)

    h0, starts_f = _tc_mlp0(x, cat3, W0, r2(b0), r2(g0), r2(beta0))
    starts = starts_f.astype(jnp.int32).reshape(128)

    a0 = _sc_segmax(h0.reshape(NP * 64), cat_pad, starts)
    p1 = _tc_tabmm(a0.reshape(C_PAD, 64), W1[64:])
    g1v = _sc_gather(p1, cat_pad)
    h1 = _tc_mid(h0, g1v, W1[:64], r2(b1), r2(g1), r2(beta1))

    a1 = _sc_segmax(h1.reshape(NP * 64), cat_pad, starts)
    p2 = _tc_tabmm(a1.reshape(C_PAD, 64), W2[64:])
    g2v = _sc_gather(p2, cat_pad)
    h2, ssh = _tc_last(h1, g2v, W2[:64], r2(b2), r2(g2), r2(beta2))

    a2 = _sc_segmax(h2.reshape(NP * 64), cat_pad, starts)
    ga2 = _sc_gather(a2.reshape(C_PAD, 64), cat_pad)
    ssg = _tc_colsq(ga2)
    return _tc_final(h2, ga2, ssh, ssg)
